# baseline (speedup 1.0000x reference)
import functools

import jax
import jax.numpy as jnp
from jax import lax
from jax.experimental import pallas as pl
from jax.experimental.pallas import tpu as pltpu
from jax.experimental.pallas import tpu_sc as plsc

_N_E = 1024
_E_DIM = 64
_BETA = 0.25

_BM = 1536
_NW = 32
_IDX_CHUNK = 96


def _distance_body(x_ref, cb_ref, d_hbm, idx_ref, loss_ref,
                   esq_ref, cbm2_ref, d_buf, sem0, sem1,
                   *, nblocks, scale):
    i = pl.program_id(0)
    sems = (sem0, sem1)

    @pl.when(i == 0)
    def _init():
        cb = cb_ref[...]
        ones = jnp.ones((1, _E_DIM), jnp.float32)
        esq_ref[...] = lax.dot_general(ones, cb * cb, (((1,), (1,)), ((), ())),
                                       preferred_element_type=jnp.float32,
                                       precision=lax.Precision.HIGHEST)
        cbm2_ref[...] = cb * -2.0

    slot = lax.rem(i, 2)

    def _copy(j, s):
        return pltpu.make_async_copy(
            d_buf.at[s], d_hbm.at[pl.ds(j * _BM, _BM), :], sems[s])

    @pl.when(i >= 2)
    def _drain():
        for s in range(2):
            @pl.when(slot == s)
            def _():
                _copy(i - 2, s).wait()

    xb = x_ref[...]
    mm2 = lax.dot_general(xb, cbm2_ref[...], (((1,), (1,)), ((), ())),
                          preferred_element_type=jnp.float32)
    zsq = jnp.sum(xb * xb, axis=1, keepdims=True)
    d = (zsq + esq_ref[...]) + mm2

    for s in range(2):
        @pl.when(slot == s)
        def _():
            d_buf[s] = d
            _copy(i, s).start()

    minv = jnp.min(d, axis=1, keepdims=True)
    iota = lax.broadcasted_iota(jnp.int32, (1, _N_E), 1).astype(jnp.float32)
    idxf = jnp.min(jnp.where(d == minv, iota, jnp.float32(2**24)), axis=1)
    idx_ref[0, 0, :] = idxf.astype(jnp.int32)

    prev = jnp.where(i == 0, jnp.zeros_like(loss_ref[...]), loss_ref[...])
    tot = prev + jnp.sum(minv)
    loss_ref[...] = jnp.where(i == nblocks - 1, tot * scale, tot)

    @pl.when(i == nblocks - 1)
    def _flush():
        for s in range(2):
            @pl.when(slot == s)
            def _():
                _copy(i - 1, 1 - s).wait()
                _copy(i, s).wait()


def _distances(latent, codebook):
    n = latent.shape[0]
    nblocks = n // _BM
    scale = (1.0 + _BETA) / float(latent.size)
    body = functools.partial(_distance_body, nblocks=nblocks, scale=scale)
    return pl.pallas_call(
        body,
        grid=(nblocks,),
        in_specs=[
            pl.BlockSpec((_BM, _E_DIM), lambda i: (i, 0)),
            pl.BlockSpec((_N_E, _E_DIM), lambda i: (0, 0)),
        ],
        out_specs=[
            pl.BlockSpec(memory_space=pl.ANY),
            pl.BlockSpec((1, 1, _BM), lambda i: (i, 0, 0)),
            pl.BlockSpec((1, 1), lambda i: (0, 0)),
        ],
        out_shape=[
            jax.ShapeDtypeStruct((n, _N_E), jnp.float32),
            jax.ShapeDtypeStruct((nblocks, 1, _BM), jnp.int32),
            jax.ShapeDtypeStruct((1, 1), jnp.float32),
        ],
        scratch_shapes=[
            pltpu.VMEM((1, _N_E), jnp.float32),
            pltpu.VMEM((_N_E, _E_DIM), jnp.float32),
            pltpu.VMEM((2, _BM, _N_E), jnp.float32),
            pltpu.SemaphoreType.DMA,
            pltpu.SemaphoreType.DMA,
        ],
    )(latent, codebook)


def _gather_codebook(codebook, idx_flat, out_shape):
    bpw = idx_flat.shape[0] // _NW
    nchunks = bpw // _IDX_CHUNK
    mesh = plsc.VectorSubcoreMesh(core_axis_name="c", subcore_axis_name="s")

    @functools.partial(
        pl.kernel, mesh=mesh,
        compiler_params=pltpu.CompilerParams(use_tc_tiling_on_sc=False),
        out_type=jax.ShapeDtypeStruct(out_shape, jnp.float32),
        scratch_types=[
            pltpu.VMEM((bpw,), jnp.int32),
            pltpu.VMEM((bpw, _E_DIM), jnp.float32),
            pltpu.SemaphoreType.DMA,
        ],
    )
    def k(cb_hbm, idx_hbm, out_hbm, idx_v, rows_v, sem):
        wid = lax.axis_index("s") * 2 + lax.axis_index("c")
        pltpu.sync_copy(idx_hbm.at[pl.ds(wid * bpw, bpw)], idx_v)
        copies = [
            pltpu.async_copy(cb_hbm.at[idx_v.at[pl.ds(j * _IDX_CHUNK, _IDX_CHUNK)]],
                             rows_v.at[pl.ds(j * _IDX_CHUNK, _IDX_CHUNK)], sem)
            for j in range(nchunks)
        ]
        for c in copies:
            c.wait()
        pltpu.sync_copy(rows_v, out_hbm.at[wid])

    return k(codebook, idx_flat)


def kernel(x, codebook):
    latent = x.reshape(-1, _E_DIM)
    d, idx3, lsum = _distances(latent, codebook)
    idx_flat = idx3.reshape(-1)
    x_q = _gather_codebook(codebook, idx_flat, x.shape)
    loss = lsum.reshape(())
    indices_out = idx_flat.reshape(x.shape[:-1])
    return (x_q, loss, indices_out, d)


# speedup vs baseline: 1.0083x; 1.0083x over previous
import functools

import jax
import jax.numpy as jnp
from jax import lax
from jax.experimental import pallas as pl
from jax.experimental.pallas import tpu as pltpu
from jax.experimental.pallas import tpu_sc as plsc

_N_E = 1024
_E_DIM = 64
_BETA = 0.25

_BM = 2304
_NW = 32
_IDX_CHUNK = 96


def _distance_body(x_ref, cb_ref, d_hbm, idx_ref, loss_ref,
                   esq_ref, cbm2_ref, d_buf, sem0, sem1,
                   *, nblocks, scale):
    i = pl.program_id(0)
    sems = (sem0, sem1)

    @pl.when(i == 0)
    def _init():
        cb = cb_ref[...]
        ones = jnp.ones((1, _E_DIM), jnp.float32)
        esq_ref[...] = lax.dot_general(ones, cb * cb, (((1,), (1,)), ((), ())),
                                       preferred_element_type=jnp.float32,
                                       precision=lax.Precision.HIGHEST)
        cbm2_ref[...] = cb * -2.0

    slot = lax.rem(i, 2)

    def _copy(j, s):
        return pltpu.make_async_copy(
            d_buf.at[s], d_hbm.at[pl.ds(j * _BM, _BM), :], sems[s])

    @pl.when(i >= 2)
    def _drain():
        for s in range(2):
            @pl.when(slot == s)
            def _():
                _copy(i - 2, s).wait()

    xb = x_ref[...]
    mm2 = lax.dot_general(xb, cbm2_ref[...], (((1,), (1,)), ((), ())),
                          preferred_element_type=jnp.float32)
    zsq = jnp.sum(xb * xb, axis=1, keepdims=True)
    d = (zsq + esq_ref[...]) + mm2

    for s in range(2):
        @pl.when(slot == s)
        def _():
            d_buf[s] = d
            _copy(i, s).start()

    minv = jnp.min(d, axis=1, keepdims=True)
    iota = lax.broadcasted_iota(jnp.int32, (1, _N_E), 1).astype(jnp.float32)
    idxf = jnp.min(jnp.where(d == minv, iota, jnp.float32(2**24)), axis=1)
    idx_ref[0, 0, :] = idxf.astype(jnp.int32)

    prev = jnp.where(i == 0, jnp.zeros_like(loss_ref[...]), loss_ref[...])
    tot = prev + jnp.sum(minv)
    loss_ref[...] = jnp.where(i == nblocks - 1, tot * scale, tot)

    @pl.when(i == nblocks - 1)
    def _flush():
        for s in range(2):
            @pl.when(slot == s)
            def _():
                _copy(i - 1, 1 - s).wait()
                _copy(i, s).wait()


def _distances(latent, codebook):
    n = latent.shape[0]
    nblocks = n // _BM
    scale = (1.0 + _BETA) / float(latent.size)
    body = functools.partial(_distance_body, nblocks=nblocks, scale=scale)
    return pl.pallas_call(
        body,
        grid=(nblocks,),
        in_specs=[
            pl.BlockSpec((_BM, _E_DIM), lambda i: (i, 0)),
            pl.BlockSpec((_N_E, _E_DIM), lambda i: (0, 0)),
        ],
        out_specs=[
            pl.BlockSpec(memory_space=pl.ANY),
            pl.BlockSpec((1, 1, _BM), lambda i: (i, 0, 0)),
            pl.BlockSpec((1, 1), lambda i: (0, 0)),
        ],
        out_shape=[
            jax.ShapeDtypeStruct((n, _N_E), jnp.float32),
            jax.ShapeDtypeStruct((nblocks, 1, _BM), jnp.int32),
            jax.ShapeDtypeStruct((1, 1), jnp.float32),
        ],
        scratch_shapes=[
            pltpu.VMEM((1, _N_E), jnp.float32),
            pltpu.VMEM((_N_E, _E_DIM), jnp.float32),
            pltpu.VMEM((2, _BM, _N_E), jnp.float32),
            pltpu.SemaphoreType.DMA,
            pltpu.SemaphoreType.DMA,
        ],
    )(latent, codebook)


def _gather_codebook(codebook, idx_flat, out_shape):
    bpw = idx_flat.shape[0] // _NW
    nchunks = bpw // _IDX_CHUNK
    mesh = plsc.VectorSubcoreMesh(core_axis_name="c", subcore_axis_name="s")

    @functools.partial(
        pl.kernel, mesh=mesh,
        compiler_params=pltpu.CompilerParams(use_tc_tiling_on_sc=False),
        out_type=jax.ShapeDtypeStruct(out_shape, jnp.float32),
        scratch_types=[
            pltpu.VMEM((bpw,), jnp.int32),
            pltpu.VMEM((bpw, _E_DIM), jnp.float32),
            pltpu.SemaphoreType.DMA,
        ],
    )
    def k(cb_hbm, idx_hbm, out_hbm, idx_v, rows_v, sem):
        wid = lax.axis_index("s") * 2 + lax.axis_index("c")
        pltpu.sync_copy(idx_hbm.at[pl.ds(wid * bpw, bpw)], idx_v)
        copies = [
            pltpu.async_copy(cb_hbm.at[idx_v.at[pl.ds(j * _IDX_CHUNK, _IDX_CHUNK)]],
                             rows_v.at[pl.ds(j * _IDX_CHUNK, _IDX_CHUNK)], sem)
            for j in range(nchunks)
        ]
        for c in copies:
            c.wait()
        pltpu.sync_copy(rows_v, out_hbm.at[wid])

    return k(codebook, idx_flat)


def kernel(x, codebook):
    latent = x.reshape(-1, _E_DIM)
    d, idx3, lsum = _distances(latent, codebook)
    idx_flat = idx3.reshape(-1)
    x_q = _gather_codebook(codebook, idx_flat, x.shape)
    loss = lsum.reshape(())
    indices_out = idx_flat.reshape(x.shape[:-1])
    return (x_q, loss, indices_out, d)
